# baseline (speedup 1.0000x reference)
import functools

import jax
import jax.numpy as jnp
from jax import lax
from jax.experimental import pallas as pl
from jax.experimental.pallas import tpu as pltpu
from jax.experimental.pallas import tpu_sc as plsc

_BATCH = 4096
_QLEN = 20
_DIM = 16
_B = _BATCH * _QLEN
_NC = 2
_NS = 16
_NW = _NC * _NS
_BPW = _B // _NW
_CH = 128
_NCH = _BPW // _CH

_mesh = plsc.VectorSubcoreMesh(core_axis_name="c", subcore_axis_name="s")


@jax.jit
def _gather(idx, table):
    @functools.partial(
        pl.kernel,
        mesh=_mesh,
        out_type=jax.ShapeDtypeStruct((_NW, _BPW, _DIM), jnp.float32),
        scratch_types=[
            pltpu.VMEM((_NCH, _CH), jnp.int32),
            pltpu.VMEM((_BPW, _DIM), jnp.float32),
            pltpu.SemaphoreType.DMA,
        ],
        compiler_params=pltpu.CompilerParams(use_tc_tiling_on_sc=False),
    )
    def body(idx_hbm, table_hbm, out_hbm, idx_v, rows_v, sem):
        wid = lax.axis_index("s") * _NC + lax.axis_index("c")
        pltpu.sync_copy(idx_hbm.at[wid], idx_v)
        copies = []
        for j in range(_NCH):
            copies.append(
                pltpu.async_copy(
                    table_hbm.at[idx_v.at[j]],
                    rows_v.at[pl.ds(j * _CH, _CH)],
                    sem,
                )
            )
        for c in copies:
            c.wait()
        pltpu.sync_copy(rows_v, out_hbm.at[wid])

    return body(idx, table)


def kernel(posdoc, query, query_idf, table):
    idx = query.reshape(_NW, _NCH, _CH)
    out = _gather(idx, table)
    return out.reshape(_BATCH, _QLEN, _DIM)


# speedup vs baseline: 1.5997x; 1.5997x over previous
import functools

import jax
import jax.numpy as jnp
from jax import lax
from jax.experimental import pallas as pl
from jax.experimental.pallas import tpu as pltpu
from jax.experimental.pallas import tpu_sc as plsc

_BATCH = 4096
_QLEN = 20
_DIM = 16
_VOCAB = 1000000
_NC = 2
_NS = 16
_NW = _NC * _NS
_BPW = _BATCH // _NW
_G = _VOCAB // 8
_KB = 8192
_KG = 123

_mesh = plsc.VectorSubcoreMesh(core_axis_name="c", subcore_axis_name="s")


def _repack_body(x_ref, o_ref):
    x = x_ref[...]
    y3 = x.T.reshape(_KB // 8, 8, 16)
    o_ref[...] = jnp.concatenate([y3[:, a, :] for a in range(8)], axis=1)


@jax.jit
def _repack(table_t):
    return pl.pallas_call(
        _repack_body,
        grid=(_KG,),
        in_specs=[pl.BlockSpec((_DIM, _KB), lambda i: (0, i))],
        out_specs=pl.BlockSpec((_KB // 8, 128), lambda i: (i, 0)),
        out_shape=jax.ShapeDtypeStruct((_G, 128), jnp.float32),
    )(table_t)


@jax.jit
def _gather(query_pad, table8):
    @functools.partial(
        pl.kernel,
        mesh=_mesh,
        out_type=jax.ShapeDtypeStruct((_QLEN, 2, 8 * _NW, _BPW), jnp.float32),
        scratch_types=[
            pltpu.VMEM((24, _BPW), jnp.int32),
            pltpu.VMEM((2, _BPW), jnp.int32),
            pltpu.VMEM((2, _BPW, 128), jnp.float32),
            pltpu.VMEM((_QLEN, 2, 8, _BPW), jnp.float32),
            pltpu.SemaphoreType.DMA,
            pltpu.SemaphoreType.DMA,
        ],
        compiler_params=pltpu.CompilerParams(
            use_tc_tiling_on_sc=True, needs_layout_passes=False
        ),
    )
    def body(qp_hbm, t8_hbm, out_hbm, idx_v, g_v, rows_v, out_v, sem0, sem1):
        wid = lax.axis_index("s") * _NC + lax.axis_index("c")
        base = wid * _BPW
        sems = (sem0, sem1)
        iota = lax.iota(jnp.int32, 16)

        for h in range(3):
            pltpu.sync_copy(
                qp_hbm.at[pl.ds(h * 8, 8), pl.ds(base, _BPW)],
                idx_v.at[pl.ds(h * 8, 8), :],
            )

        def fire(q, buf):
            for k in range(8):
                idx16 = idx_v[q, pl.ds(k * 16, 16)]
                g_v[buf, pl.ds(k * 16, 16)] = lax.shift_right_logical(idx16, 3)
            return pltpu.async_copy(
                t8_hbm.at[g_v.at[buf]], rows_v.at[buf], sems[buf]
            )

        def wait_chunk(buf):
            pltpu.make_async_copy(
                t8_hbm.at[pl.ds(0, _BPW), :], rows_v.at[buf], sems[buf]
            ).wait()

        def extract(q, buf):
            for k in range(8):
                idx16 = idx_v[q, pl.ds(k * 16, 16)]
                colbase = lax.shift_left(
                    lax.bitwise_and(idx16, jnp.int32(7)), 4
                )
                rowids = iota + jnp.int32(k * 16)
                for t in range(16):
                    vals = plsc.load_gather(
                        rows_v.at[buf], [rowids, colbase + jnp.int32(t)]
                    )
                    out_v[q, t // 8, t % 8, pl.ds(k * 16, 16)] = vals

        fire(0, 0)

        @pl.loop(0, _QLEN, step=2)
        def _(q):
            fire(q + 1, 1)
            wait_chunk(0)
            extract(q, 0)

            @pl.when(q + 2 < _QLEN)
            def _():
                fire(q + 2, 0)

            wait_chunk(1)
            extract(q + 1, 1)

        pltpu.sync_copy(out_v, out_hbm.at[:, :, pl.ds(wid * 8, 8), :])

    return body(query_pad, table8)


def kernel(posdoc, query, query_idf, table):
    qpad = jnp.pad(query.T, ((0, 4), (0, 0)))
    table8 = _repack(table.T)
    out5 = _gather(qpad, table8)
    out = out5.reshape(_QLEN, 2, _NW, 8, _BPW)
    out = out.transpose(2, 4, 0, 1, 3)
    return out.reshape(_BATCH, _QLEN, _DIM)


# speedup vs baseline: 1.8166x; 1.1356x over previous
import functools

import jax
import jax.numpy as jnp
from jax import lax
from jax.experimental import pallas as pl
from jax.experimental.pallas import tpu as pltpu
from jax.experimental.pallas import tpu_sc as plsc

_BATCH = 4096
_QLEN = 20
_DIM = 16
_VOCAB = 1000000
_NC = 2
_NS = 16
_NW = _NC * _NS
_BPW = _BATCH // _NW
_G = _VOCAB // 8
_KB = 8192
_KG = 123

_mesh = plsc.VectorSubcoreMesh(core_axis_name="c", subcore_axis_name="s")


def _repack_body(x_ref, o_ref):
    x = x_ref[...]
    y3 = x.T.reshape(_KB // 8, 8, 16)
    for a in range(8):
        o_ref[:, a * 16:(a + 1) * 16] = y3[:, a, :]


@jax.jit
def _repack(table_t):
    return pl.pallas_call(
        _repack_body,
        grid=(_KG,),
        in_specs=[pl.BlockSpec((_DIM, _KB), lambda i: (0, i))],
        out_specs=pl.BlockSpec((_KB // 8, 128), lambda i: (i, 0)),
        out_shape=jax.ShapeDtypeStruct((_G, 128), jnp.float32),
    )(table_t)


@jax.jit
def _gather(query_pad, table8):
    @functools.partial(
        pl.kernel,
        mesh=_mesh,
        out_type=jax.ShapeDtypeStruct((_QLEN, 2, 8 * _NW, _BPW), jnp.float32),
        scratch_types=[
            pltpu.VMEM((24, _BPW), jnp.int32),
            pltpu.VMEM((2, _BPW), jnp.int32),
            pltpu.VMEM((2, _BPW, 128), jnp.float32),
            pltpu.VMEM((_QLEN, 2, 8, _BPW), jnp.float32),
            pltpu.SemaphoreType.DMA,
            pltpu.SemaphoreType.DMA,
        ],
        compiler_params=pltpu.CompilerParams(
            use_tc_tiling_on_sc=True, needs_layout_passes=False
        ),
    )
    def body(qp_hbm, t8_hbm, out_hbm, idx_v, g_v, rows_v, out_v, sem0, sem1):
        wid = lax.axis_index("s") * _NC + lax.axis_index("c")
        base = wid * _BPW
        sems = (sem0, sem1)
        iota = lax.iota(jnp.int32, 16)

        for h in range(3):
            pltpu.sync_copy(
                qp_hbm.at[pl.ds(h * 8, 8), pl.ds(base, _BPW)],
                idx_v.at[pl.ds(h * 8, 8), :],
            )

        def fire(q, buf):
            for k in range(8):
                idx16 = idx_v[q, pl.ds(k * 16, 16)]
                g_v[buf, pl.ds(k * 16, 16)] = lax.shift_right_logical(idx16, 3)
            return pltpu.async_copy(
                t8_hbm.at[g_v.at[buf]], rows_v.at[buf], sems[buf]
            )

        def wait_chunk(buf):
            pltpu.make_async_copy(
                t8_hbm.at[pl.ds(0, _BPW), :], rows_v.at[buf], sems[buf]
            ).wait()

        def extract(q, buf):
            for k in range(8):
                idx16 = idx_v[q, pl.ds(k * 16, 16)]
                colbase = lax.shift_left(
                    lax.bitwise_and(idx16, jnp.int32(7)), 4
                )
                rowids = iota + jnp.int32(k * 16)
                for t in range(16):
                    vals = plsc.load_gather(
                        rows_v.at[buf], [rowids, colbase + jnp.int32(t)]
                    )
                    out_v[q, t // 8, t % 8, pl.ds(k * 16, 16)] = vals

        fire(0, 0)

        @pl.loop(0, _QLEN, step=2)
        def _(q):
            fire(q + 1, 1)
            wait_chunk(0)
            extract(q, 0)

            @pl.when(q + 2 < _QLEN)
            def _():
                fire(q + 2, 0)

            wait_chunk(1)
            extract(q + 1, 1)

        pltpu.sync_copy(out_v, out_hbm.at[:, :, pl.ds(wid * 8, 8), :])

    return body(query_pad, table8)


def kernel(posdoc, query, query_idf, table):
    qpad = jnp.pad(query.T, ((0, 4), (0, 0)))
    table8 = _repack(table.T)
    out5 = _gather(qpad, table8)
    out = out5.reshape(_QLEN, 2, _NW, 8, _BPW)
    out = out.transpose(2, 4, 0, 1, 3)
    return out.reshape(_BATCH, _QLEN, _DIM)
